# baseline (speedup 1.0000x reference)
import functools

import jax
import jax.numpy as jnp
from jax import lax
from jax.experimental import pallas as pl
from jax.experimental.pallas import tpu as pltpu
from jax.experimental.pallas import tpu_sc as plsc

N = 10000
E = 320000
D = 128
G = 16
NC, NS = 2, 16
NW = NC * NS
CH = 128
NCH = 80
E_W = NCH * CH
E_PAD = NW * E_W
NP = N + 16

_mesh = plsc.VectorSubcoreMesh(core_axis_name="c", subcore_axis_name="s")


@functools.partial(
    pl.kernel,
    mesh=_mesh,
    out_type=jax.ShapeDtypeStruct((NC * N,), jnp.float32),
    scratch_types=[
        pltpu.VMEM((NCH, CH), jnp.int32),
        pltpu.VMEM((CH,), jnp.float32),
        pltpu.VMEM((N,), jnp.float32),
        pltpu.VMEM_SHARED((NP,), jnp.float32),
        pltpu.SemaphoreType.DMA,
    ],
)
def _deg_kernel(dst_hbm, zero_hbm, out_hbm, didx_v, ones_v, bounce_v, acc_s,
                sem):
    c = lax.axis_index("c")
    s = lax.axis_index("s")
    w = c * NS + s
    for k in range(CH // 16):
        ones_v[pl.ds(k * 16, 16)] = jnp.ones((16,), jnp.float32)
    pltpu.sync_copy(dst_hbm.at[pl.ds(pl.multiple_of(w * NCH, 8), NCH)],
                    didx_v)

    @pl.when(s == 0)
    def _():
        pltpu.sync_copy(zero_hbm, bounce_v)
        pltpu.sync_copy(bounce_v, acc_s.at[pl.ds(0, N)])
        pltpu.sync_copy(bounce_v.at[pl.ds(0, 16)], acc_s.at[pl.ds(N, 16)])

    plsc.subcore_barrier()

    KG = 8

    def body(t, carry):
        for b in range(KG):
            pltpu.async_copy(ones_v, acc_s.at[didx_v.at[t * KG + b]], sem,
                             add=True)
        for b in range(KG):
            pltpu.make_async_copy(ones_v, acc_s.at[didx_v.at[t * KG + b]],
                                  sem).wait()
        return carry

    lax.fori_loop(0, NCH // KG, body, 0)
    plsc.subcore_barrier()

    @pl.when(s == 0)
    def _():
        pltpu.sync_copy(acc_s.at[pl.ds(0, N)], bounce_v)
        pltpu.sync_copy(bounce_v,
                        out_hbm.at[pl.ds(pl.multiple_of(c * N, 8), N)])


@functools.partial(
    pl.kernel,
    mesh=_mesh,
    out_type=jax.ShapeDtypeStruct((NC, N, D), jnp.float32),
    scratch_types=[
        pltpu.VMEM((NCH // 2, CH), jnp.int32),
        pltpu.VMEM((NCH // 2, CH), jnp.int32),
        pltpu.VMEM((CH, D), jnp.float32),
        pltpu.VMEM((CH, D), jnp.float32),
        pltpu.VMEM_SHARED((N, D), jnp.float32),
        pltpu.SemaphoreType.DMA,
        pltpu.SemaphoreType.DMA,
    ],
)
def _agg_kernel(xs_hbm, src_hbm, dst_hbm, zero_hbm, out_hbm, sidx_v, didx_v,
                rows0_v, rows1_v, acc_s, sem0, sem1):
    c = lax.axis_index("c")
    s = lax.axis_index("s")
    w = c * NS + s
    HCH = NCH // 2

    @pl.when(s == 0)
    def _():
        pltpu.sync_copy(zero_hbm, acc_s)

    plsc.subcore_barrier()

    for h in range(2):
        base = pl.multiple_of(w * NCH + h * HCH, 8)
        pltpu.sync_copy(src_hbm.at[pl.ds(base, HCH)], sidx_v)
        pltpu.sync_copy(dst_hbm.at[pl.ds(base, HCH)], didx_v)
        pltpu.async_copy(xs_hbm.at[sidx_v.at[0]], rows0_v, sem0)

        def body(i, carry):
            g0 = i * 2
            g1 = g0 + 1
            pltpu.async_copy(xs_hbm.at[sidx_v.at[g1]], rows1_v, sem1)
            pltpu.make_async_copy(xs_hbm.at[sidx_v.at[g0]], rows0_v,
                                  sem0).wait()
            pltpu.sync_copy(rows0_v, acc_s.at[didx_v.at[g0]], add=True)

            @pl.when(i < HCH // 2 - 1)
            def _():
                pltpu.async_copy(xs_hbm.at[sidx_v.at[g0 + 2]], rows0_v, sem0)

            pltpu.make_async_copy(xs_hbm.at[sidx_v.at[g1]], rows1_v,
                                  sem1).wait()
            pltpu.sync_copy(rows1_v, acc_s.at[didx_v.at[g1]], add=True)
            return carry

        lax.fori_loop(0, HCH // 2, body, 0)
    plsc.subcore_barrier()

    @pl.when(s == 0)
    def _():
        pltpu.sync_copy(acc_s, out_hbm.at[c])


def _scale_body(deg_ref, x_ref, xs_ref):
    dinv = lax.rsqrt(deg_ref[:, 0:1] + deg_ref[:, 1:2] + 1.0)
    xs_ref[0:N, :] = x_ref[...] * dinv
    xs_ref[N:NP, :] = jnp.zeros((NP - N, D), jnp.float32)


def _layer_body(acc_ref, deg_ref, xin_ref, w_ref, b_ref, h_ref, xs_ref):
    dinv = lax.rsqrt(deg_ref[:, 0:1] + deg_ref[:, 1:2] + 1.0)
    agg = dinv * (acc_ref[0] + acc_ref[1]) + (dinv * dinv) * xin_ref[...]
    h = jnp.dot(agg, w_ref[...], preferred_element_type=jnp.float32)
    h = jnp.maximum(h + b_ref[...], 0.0)
    h_ref[...] = h
    xs_ref[0:N, :] = h * dinv
    xs_ref[N:NP, :] = jnp.zeros((NP - N, D), jnp.float32)


def _final_body(acc_ref, deg_ref, h1_ref, w_ref, b_ref, batch_ref, lw_ref,
                lb_ref, out_ref):
    dinv = lax.rsqrt(deg_ref[:, 0:1] + deg_ref[:, 1:2] + 1.0)
    agg = dinv * (acc_ref[0] + acc_ref[1]) + (dinv * dinv) * h1_ref[...]
    h2 = jnp.dot(agg, w_ref[...], preferred_element_type=jnp.float32)
    h2 = jnp.maximum(h2 + b_ref[...], 0.0)
    gid = lax.broadcasted_iota(jnp.int32, (G, N), 0)
    onehot = (jnp.broadcast_to(batch_ref[...], (G, N)) == gid)
    onehot = onehot.astype(jnp.float32)
    sums = jnp.dot(onehot, h2, preferred_element_type=jnp.float32)
    counts = jnp.sum(onehot, axis=1, keepdims=True)
    pooled = sums / jnp.maximum(counts, 1.0)
    out_ref[...] = (
        jnp.dot(pooled, lw_ref[...], preferred_element_type=jnp.float32)
        + lb_ref[...])


_scale_call = pl.pallas_call(
    _scale_body, out_shape=jax.ShapeDtypeStruct((NP, D), jnp.float32))

_layer_call = pl.pallas_call(
    _layer_body,
    out_shape=(jax.ShapeDtypeStruct((N, D), jnp.float32),
               jax.ShapeDtypeStruct((NP, D), jnp.float32)))

_final_call = pl.pallas_call(
    _final_body, out_shape=jax.ShapeDtypeStruct((G, 1), jnp.float32))


@jax.jit
def kernel(x, edge_index, batch, W1, b1, W2, b2, lin_W, lin_b):
    src = edge_index[0]
    dst = edge_index[1]
    pad = E_PAD - E
    src2d = jnp.concatenate([src, jnp.full((pad,), N, jnp.int32)]).reshape(
        NW * NCH, CH)
    dst2d_deg = jnp.concatenate([dst, jnp.full((pad,), N, jnp.int32)]).reshape(
        NW * NCH, CH)
    dst2d = jnp.concatenate([dst, jnp.zeros((pad,), jnp.int32)]).reshape(
        NW * NCH, CH)
    zero1 = jnp.zeros((N,), jnp.float32)
    zero2 = jnp.zeros((N, D), jnp.float32)
    deg_t = _deg_kernel(dst2d_deg, zero1).reshape(NC, N).T
    xs1 = _scale_call(deg_t, x)
    acc1 = _agg_kernel(xs1, src2d, dst2d, zero2)
    h1, xs2 = _layer_call(acc1, deg_t, x, W1, b1)
    acc2 = _agg_kernel(xs2, src2d, dst2d, zero2)
    return _final_call(acc2, deg_t, h1, W2, b2, batch.reshape(1, N),
                       lin_W, lin_b)


# speedup vs baseline: 1.0060x; 1.0060x over previous
import functools

import jax
import jax.numpy as jnp
from jax import lax
from jax.experimental import pallas as pl
from jax.experimental.pallas import tpu as pltpu
from jax.experimental.pallas import tpu_sc as plsc

N = 10000
E = 320000
D = 128
G = 16
NC, NS = 2, 16
NW = NC * NS
CH = 128
NCH = 80
E_W = NCH * CH
E_PAD = NW * E_W
NP = N + 16
NPD = N + 640

_mesh = plsc.VectorSubcoreMesh(core_axis_name="c", subcore_axis_name="s")


@functools.partial(
    pl.kernel,
    mesh=_mesh,
    out_type=jax.ShapeDtypeStruct((NC * N,), jnp.float32),
    scratch_types=[
        pltpu.VMEM((NCH, CH), jnp.int32),
        pltpu.VMEM((CH,), jnp.float32),
        pltpu.VMEM((N,), jnp.float32),
        pltpu.VMEM_SHARED((NPD,), jnp.float32),
        pltpu.SemaphoreType.DMA,
    ],
)
def _deg_kernel(dst_hbm, zero_hbm, out_hbm, didx_v, ones_v, bounce_v, acc_s,
                sem):
    c = lax.axis_index("c")
    s = lax.axis_index("s")
    w = c * NS + s
    for k in range(CH // 16):
        ones_v[pl.ds(k * 16, 16)] = jnp.ones((16,), jnp.float32)
    pltpu.sync_copy(dst_hbm.at[pl.ds(pl.multiple_of(w * NCH, 8), NCH)],
                    didx_v)

    @pl.when(s == 0)
    def _():
        pltpu.sync_copy(zero_hbm, bounce_v)
        pltpu.sync_copy(bounce_v, acc_s.at[pl.ds(0, N)])
        pltpu.sync_copy(bounce_v.at[pl.ds(0, NPD - N)],
                        acc_s.at[pl.ds(N, NPD - N)])

    plsc.subcore_barrier()

    KG = 8

    def body(t, carry):
        for b in range(KG):
            pltpu.async_copy(ones_v, acc_s.at[didx_v.at[t * KG + b]], sem,
                             add=True)
        for b in range(KG):
            pltpu.make_async_copy(ones_v, acc_s.at[didx_v.at[t * KG + b]],
                                  sem).wait()
        return carry

    lax.fori_loop(0, NCH // KG, body, 0)
    plsc.subcore_barrier()

    @pl.when(s == 0)
    def _():
        pltpu.sync_copy(acc_s.at[pl.ds(0, N)], bounce_v)
        pltpu.sync_copy(bounce_v,
                        out_hbm.at[pl.ds(pl.multiple_of(c * N, 8), N)])


@functools.partial(
    pl.kernel,
    mesh=_mesh,
    out_type=jax.ShapeDtypeStruct((NC, N, D), jnp.float32),
    scratch_types=[
        pltpu.VMEM((NCH // 2, CH), jnp.int32),
        pltpu.VMEM((NCH // 2, CH), jnp.int32),
        pltpu.VMEM((CH, D), jnp.float32),
        pltpu.VMEM((CH, D), jnp.float32),
        pltpu.VMEM_SHARED((N, D), jnp.float32),
        pltpu.SemaphoreType.DMA,
        pltpu.SemaphoreType.DMA,
    ],
)
def _agg_kernel(xs_hbm, src_hbm, dst_hbm, zero_hbm, out_hbm, sidx_v, didx_v,
                rows0_v, rows1_v, acc_s, sem0, sem1):
    c = lax.axis_index("c")
    s = lax.axis_index("s")
    w = c * NS + s
    HCH = NCH // 2

    @pl.when(s == 0)
    def _():
        pltpu.sync_copy(zero_hbm, acc_s)

    plsc.subcore_barrier()

    for h in range(2):
        base = pl.multiple_of(w * NCH + h * HCH, 8)
        pltpu.sync_copy(src_hbm.at[pl.ds(base, HCH)], sidx_v)
        pltpu.sync_copy(dst_hbm.at[pl.ds(base, HCH)], didx_v)
        pltpu.async_copy(xs_hbm.at[sidx_v.at[0]], rows0_v, sem0)

        def body(i, carry):
            g0 = i * 2
            g1 = g0 + 1
            pltpu.async_copy(xs_hbm.at[sidx_v.at[g1]], rows1_v, sem1)
            pltpu.make_async_copy(xs_hbm.at[sidx_v.at[g0]], rows0_v,
                                  sem0).wait()
            pltpu.sync_copy(rows0_v, acc_s.at[didx_v.at[g0]], add=True)

            @pl.when(i < HCH // 2 - 1)
            def _():
                pltpu.async_copy(xs_hbm.at[sidx_v.at[g0 + 2]], rows0_v, sem0)

            pltpu.make_async_copy(xs_hbm.at[sidx_v.at[g1]], rows1_v,
                                  sem1).wait()
            pltpu.sync_copy(rows1_v, acc_s.at[didx_v.at[g1]], add=True)
            return carry

        lax.fori_loop(0, HCH // 2, body, 0)
    plsc.subcore_barrier()

    @pl.when(s == 0)
    def _():
        pltpu.sync_copy(acc_s, out_hbm.at[c])


def _scale_body(deg_ref, x_ref, xs_ref):
    dinv = lax.rsqrt(deg_ref[:, 0:1] + deg_ref[:, 1:2] + 1.0)
    xs_ref[0:N, :] = x_ref[...] * dinv
    xs_ref[N:NP, :] = jnp.zeros((NP - N, D), jnp.float32)


def _layer_body(acc_ref, deg_ref, xin_ref, w_ref, b_ref, h_ref, xs_ref):
    dinv = lax.rsqrt(deg_ref[:, 0:1] + deg_ref[:, 1:2] + 1.0)
    agg = dinv * (acc_ref[0] + acc_ref[1]) + (dinv * dinv) * xin_ref[...]
    h = jnp.dot(agg, w_ref[...], preferred_element_type=jnp.float32)
    h = jnp.maximum(h + b_ref[...], 0.0)
    h_ref[...] = h
    xs_ref[0:N, :] = h * dinv
    xs_ref[N:NP, :] = jnp.zeros((NP - N, D), jnp.float32)


def _final_body(acc_ref, deg_ref, h1_ref, w_ref, b_ref, batch_ref, lw_ref,
                lb_ref, out_ref):
    dinv = lax.rsqrt(deg_ref[:, 0:1] + deg_ref[:, 1:2] + 1.0)
    agg = dinv * (acc_ref[0] + acc_ref[1]) + (dinv * dinv) * h1_ref[...]
    h2 = jnp.dot(agg, w_ref[...], preferred_element_type=jnp.float32)
    h2 = jnp.maximum(h2 + b_ref[...], 0.0)
    gid = lax.broadcasted_iota(jnp.int32, (G, N), 0)
    onehot = (jnp.broadcast_to(batch_ref[...], (G, N)) == gid)
    onehot = onehot.astype(jnp.float32)
    sums = jnp.dot(onehot, h2, preferred_element_type=jnp.float32)
    counts = jnp.sum(onehot, axis=1, keepdims=True)
    pooled = sums / jnp.maximum(counts, 1.0)
    out_ref[...] = (
        jnp.dot(pooled, lw_ref[...], preferred_element_type=jnp.float32)
        + lb_ref[...])


_scale_call = pl.pallas_call(
    _scale_body, out_shape=jax.ShapeDtypeStruct((NP, D), jnp.float32))

_layer_call = pl.pallas_call(
    _layer_body,
    out_shape=(jax.ShapeDtypeStruct((N, D), jnp.float32),
               jax.ShapeDtypeStruct((NP, D), jnp.float32)))

_final_call = pl.pallas_call(
    _final_body, out_shape=jax.ShapeDtypeStruct((G, 1), jnp.float32))


@jax.jit
def kernel(x, edge_index, batch, W1, b1, W2, b2, lin_W, lin_b):
    src = edge_index[0]
    dst = edge_index[1]
    pad = E_PAD - E
    spread = jnp.arange(pad, dtype=jnp.int32)
    src2d = jnp.concatenate([src, jnp.full((pad,), N, jnp.int32)]).reshape(
        NW * NCH, CH)
    dst2d_deg = jnp.concatenate([dst, N + spread % (NPD - N)]).reshape(
        NW * NCH, CH)
    dst2d = jnp.concatenate([dst, spread % N]).reshape(NW * NCH, CH)
    zero1 = jnp.zeros((N,), jnp.float32)
    zero2 = jnp.zeros((N, D), jnp.float32)
    deg_t = _deg_kernel(dst2d_deg, zero1).reshape(NC, N).T
    xs1 = _scale_call(deg_t, x)
    acc1 = _agg_kernel(xs1, src2d, dst2d, zero2)
    h1, xs2 = _layer_call(acc1, deg_t, x, W1, b1)
    acc2 = _agg_kernel(xs2, src2d, dst2d, zero2)
    return _final_call(acc2, deg_t, h1, W2, b2, batch.reshape(1, N),
                       lin_W, lin_b)
